# baseline (speedup 1.0000x reference)
import functools
import jax
import jax.numpy as jnp
from jax import lax
from jax.experimental import pallas as pl
from jax.experimental.pallas import tpu as pltpu
from jax.experimental.pallas import tpu_sc as plsc

S, D = 8192, 1024
NC = 2
ROWS_W = S // NC
CH = 512
NB = 3
NCHUNK = ROWS_W // CH


def _sc_body(w_hbm, o_hbm, buf, in_sems, out_sems):
    base = lax.axis_index("c") * ROWS_W

    def in_copy(g, b):
        return pltpu.make_async_copy(
            w_hbm.at[pl.ds(base + g * CH, CH)], buf.at[b], in_sems.at[b])

    def out_copy(g, b):
        return pltpu.make_async_copy(
            buf.at[b], o_hbm.at[pl.ds(base + g * CH, CH)], out_sems.at[b])

    for b in range(NB):
        in_copy(b, b).start()
    for g in range(NCHUNK):
        b = g % NB
        in_copy(g, b).wait()
        out_copy(g, b).start()
        if g + NB < NCHUNK:
            out_copy(g, b).wait()
            in_copy(g + NB, b).start()
    for g in range(NCHUNK - NB, NCHUNK):
        out_copy(g, g % NB).wait()


@jax.jit
def kernel(x, emb_weight):
    del x
    mesh = plsc.ScalarSubcoreMesh(axis_name="c", num_cores=NC)
    f = functools.partial(
        pl.kernel,
        out_type=jax.ShapeDtypeStruct((S, D), jnp.float32),
        mesh=mesh,
        scratch_types=[
            pltpu.VMEM_SHARED((NB, CH, D), jnp.float32),
            pltpu.SemaphoreType.DMA((NB,)),
            pltpu.SemaphoreType.DMA((NB,)),
        ],
    )(_sc_body)
    return f(emb_weight)


# speedup vs baseline: 1.0763x; 1.0763x over previous
import functools
import jax
import jax.numpy as jnp
from jax import lax
from jax.experimental import pallas as pl
from jax.experimental.pallas import tpu as pltpu
from jax.experimental.pallas import tpu_sc as plsc

S, D = 8192, 1024
NC, NS = 2, 16
NW = NC * NS
ROWS_W = S // NW
CH = 32
NB = 3
NCHUNK = ROWS_W // CH


def _sc_body(w_hbm, o_hbm, buf, in_sems, out_sems):
    wid = lax.axis_index("s") * NC + lax.axis_index("c")
    base = wid * ROWS_W

    def in_copy(g, b):
        return pltpu.make_async_copy(
            w_hbm.at[pl.ds(base + g * CH, CH)], buf.at[b], in_sems.at[b])

    def out_copy(g, b):
        return pltpu.make_async_copy(
            buf.at[b], o_hbm.at[pl.ds(base + g * CH, CH)], out_sems.at[b])

    for b in range(NB):
        in_copy(b, b).start()
    for g in range(NCHUNK):
        b = g % NB
        in_copy(g, b).wait()
        out_copy(g, b).start()
        if g + NB < NCHUNK:
            out_copy(g, b).wait()
            in_copy(g + NB, b).start()
    for g in range(NCHUNK - NB, NCHUNK):
        out_copy(g, g % NB).wait()


@jax.jit
def kernel(x, emb_weight):
    del x
    mesh = plsc.VectorSubcoreMesh(core_axis_name="c", subcore_axis_name="s")
    f = functools.partial(
        pl.kernel,
        out_type=jax.ShapeDtypeStruct((S, D), jnp.float32),
        mesh=mesh,
        scratch_types=[
            pltpu.VMEM((NB, CH, D), jnp.float32),
            pltpu.SemaphoreType.DMA((NB,)),
            pltpu.SemaphoreType.DMA((NB,)),
        ],
    )(_sc_body)
    return f(emb_weight)


# speedup vs baseline: 1.0798x; 1.0033x over previous
import functools
import jax
import jax.numpy as jnp
from jax import lax
from jax.experimental import pallas as pl
from jax.experimental.pallas import tpu as pltpu
from jax.experimental.pallas import tpu_sc as plsc

S, D = 8192, 1024
NC, NS = 2, 16
NW = NC * NS
ROWS_W = S // NW
CH = 32
NB = 3
NCHUNK = ROWS_W // CH


def _sc_body(w_hbm, o_hbm, buf, in_sems, out_sems):
    wid = lax.axis_index("c") * NS + lax.axis_index("s")
    base = wid * ROWS_W

    def in_copy(g, b):
        return pltpu.make_async_copy(
            w_hbm.at[pl.ds(base + g * CH, CH)], buf.at[b], in_sems.at[b])

    def out_copy(g, b):
        return pltpu.make_async_copy(
            buf.at[b], o_hbm.at[pl.ds(base + g * CH, CH)], out_sems.at[b])

    for b in range(NB):
        in_copy(b, b).start()
    for g in range(NCHUNK):
        b = g % NB
        in_copy(g, b).wait()
        out_copy(g, b).start()
        if g + NB < NCHUNK:
            out_copy(g, b).wait()
            in_copy(g + NB, b).start()
    for g in range(NCHUNK - NB, NCHUNK):
        out_copy(g, g % NB).wait()


@jax.jit
def kernel(x, emb_weight):
    del x
    mesh = plsc.VectorSubcoreMesh(core_axis_name="c", subcore_axis_name="s")
    f = functools.partial(
        pl.kernel,
        out_type=jax.ShapeDtypeStruct((S, D), jnp.float32),
        mesh=mesh,
        scratch_types=[
            pltpu.VMEM((NB, CH, D), jnp.float32),
            pltpu.SemaphoreType.DMA((NB,)),
            pltpu.SemaphoreType.DMA((NB,)),
        ],
    )(_sc_body)
    return f(emb_weight)


# speedup vs baseline: 1.0949x; 1.0139x over previous
import functools
import jax
import jax.numpy as jnp
from jax import lax
from jax.experimental import pallas as pl
from jax.experimental.pallas import tpu as pltpu
from jax.experimental.pallas import tpu_sc as plsc

S, D = 8192, 1024
NC, NS = 2, 16
NW = NC * NS
ROWS_W = S // NW
CH = 32
NB = 2
NCH_SUB = ROWS_W // CH // 2


def _stream(hbm_in, hbm_out, buf, in_sems, out_sems, base):
    def in_copy(g, b):
        return pltpu.make_async_copy(
            hbm_in.at[pl.ds(base + g * CH, CH)], buf.at[b], in_sems.at[b])

    def out_copy(g, b):
        return pltpu.make_async_copy(
            buf.at[b], hbm_out.at[pl.ds(base + g * CH, CH)], out_sems.at[b])

    return in_copy, out_copy


def _sc_body(w_hbm, o_hbm, buf_t, buf_s, ti_sems, to_sems, si_sems, so_sems):
    sid = lax.axis_index("s")
    wid = lax.axis_index("c") * NS + sid
    base_t = wid * ROWS_W
    base_s = base_t + NCH_SUB * CH

    t_in, t_out = _stream(w_hbm, o_hbm, buf_t, ti_sems, to_sems, base_t)
    s_in, s_out = _stream(w_hbm, o_hbm, buf_s.at[sid], si_sems, so_sems, base_s)

    for b in range(NB):
        t_in(b, b).start()
        s_in(b, b).start()
    for g in range(NCH_SUB):
        b = g % NB
        t_in(g, b).wait()
        t_out(g, b).start()
        s_in(g, b).wait()
        s_out(g, b).start()
        if g + NB < NCH_SUB:
            t_out(g, b).wait()
            t_in(g + NB, b).start()
            s_out(g, b).wait()
            s_in(g + NB, b).start()
    for g in range(NCH_SUB - NB, NCH_SUB):
        t_out(g, g % NB).wait()
        s_out(g, g % NB).wait()


@jax.jit
def kernel(x, emb_weight):
    del x
    mesh = plsc.VectorSubcoreMesh(core_axis_name="c", subcore_axis_name="s")
    f = functools.partial(
        pl.kernel,
        out_type=jax.ShapeDtypeStruct((S, D), jnp.float32),
        mesh=mesh,
        scratch_types=[
            pltpu.VMEM((NB, CH, D), jnp.float32),
            pltpu.VMEM_SHARED((NS, NB, CH, D), jnp.float32),
            pltpu.SemaphoreType.DMA((NB,)),
            pltpu.SemaphoreType.DMA((NB,)),
            pltpu.SemaphoreType.DMA((NB,)),
            pltpu.SemaphoreType.DMA((NB,)),
        ],
    )(_sc_body)
    return f(emb_weight)
